# baseline (speedup 1.0000x reference)
import functools

import jax
import jax.numpy as jnp
import numpy as np
from jax import lax
from jax.experimental import pallas as pl
from jax.experimental.pallas import tpu as pltpu
from jax.experimental.pallas import tpu_sc as plsc

N = 16384
D = 256
H = 128
O = 128
E = 4
EP = 8
BN_S = 1.0 / np.sqrt(1.0 + 1e-5)

TN = 2048
TNL = 2048

NSC, NSUB, L = 2, 16, 16
NW = NSC * NSUB
TOK_W = N // NW
GRP = TOK_W // L


def _dot_t(a, b):
    return jax.lax.dot_general(a, b, (((1,), (1,)), ((), ())),
                               preferred_element_type=jnp.float32)


def _sigmoid(t):
    return 1.0 / (1.0 + jnp.exp(-t))


def _logits_body(x_ref, Wg_ref, lt_ref):
    lt_ref[...] = jax.lax.dot_general(
        Wg_ref[...], x_ref[...], (((1,), (1,)), ((), ())),
        preferred_element_type=jnp.float32)


_sc_mesh = plsc.VectorSubcoreMesh(core_axis_name="c", subcore_axis_name="s")


@functools.partial(
    pl.kernel,
    mesh=_sc_mesh,
    out_type=jax.ShapeDtypeStruct((EP, N), jnp.float32),
    scratch_types=[
        pltpu.VMEM((E, TOK_W), jnp.float32),
        pltpu.VMEM((EP, TOK_W), jnp.float32),
    ],
)
def _router_sc(lt_hbm, wt_hbm, lv, wv):
    wid = lax.axis_index("s") * NSC + lax.axis_index("c")
    base = wid * TOK_W
    for e in range(E):
        pltpu.sync_copy(lt_hbm.at[e, pl.ds(base, TOK_W)], lv.at[e])
    for g in range(GRP):
        sl = pl.ds(g * L, L)
        v0 = lv[0, sl]
        v1 = lv[1, sl]
        v2 = lv[2, sl]
        v3 = lv[3, sl]
        m1 = jnp.maximum(jnp.maximum(v0, v1), jnp.maximum(v2, v3))
        i1 = jnp.where(v0 == m1, 0,
                       jnp.where(v1 == m1, 1,
                                 jnp.where(v2 == m1, 2, 3)))
        ninf = jnp.float32(-jnp.inf)
        u0 = jnp.where(i1 == 0, ninf, v0)
        u1 = jnp.where(i1 == 1, ninf, v1)
        u2 = jnp.where(i1 == 2, ninf, v2)
        u3 = jnp.where(i1 == 3, ninf, v3)
        m2 = jnp.maximum(jnp.maximum(u0, u1), jnp.maximum(u2, u3))
        i2 = jnp.where(u0 == m2, 0,
                       jnp.where(u1 == m2, 1,
                                 jnp.where(u2 == m2, 2, 3)))
        z = jnp.exp(m2 - m1)
        w1 = 1.0 / (1.0 + z)
        w2 = z * w1
        zero = jnp.float32(0.0)
        for e in range(E):
            wv[e, sl] = jnp.where(i1 == e, w1,
                                  jnp.where(i2 == e, w2, zero))
        for e in range(E, EP):
            wv[e, sl] = jnp.zeros((L,), jnp.float32)
    for e in range(EP):
        pltpu.sync_copy(wv.at[e], wt_hbm.at[e, pl.ds(base, TOK_W)])


def _moe_body(x_ref, wt_ref, W1c_ref, b1c_ref, Weh_ref, beh_ref,
              Weo_ref, beo_ref, Wsh_ref, bsh_ref, Wso_ref, bso_ref, o_ref):
    x = x_ref[...]
    w = jnp.transpose(wt_ref[...])

    hc = jnp.maximum(_dot_t(x, W1c_ref[...]) + b1c_ref[...], 0.0)

    acc = jnp.zeros((x.shape[0], O), jnp.float32)
    for e in range(E):
        he = hc[:, e * H:(e + 1) * H]
        h = jnp.maximum(_dot_t(he, Weh_ref[e]) + beh_ref[e], 0.0)
        o = _sigmoid(_dot_t(h, Weo_ref[e]) + beo_ref[e])
        acc = acc + o * w[:, e:e + 1]

    hs = hc[:, E * H:(E + 1) * H]
    h = jnp.maximum(_dot_t(hs, Wsh_ref[...]) + bsh_ref[...], 0.0)
    sf = _sigmoid(_dot_t(h, Wso_ref[...]) + bso_ref[...])

    o_ref[...] = acc + sf


@jax.jit
def _run(combined, Wg, We1, be1, Weh, beh, Weo, beo,
         Ws1, bs1, Wsh, bsh, Wso, bso):
    W1c = jnp.concatenate([We1.reshape(E * H, D), Ws1], axis=0) * BN_S
    b1c = jnp.concatenate(
        [be1.reshape(1, E * H), bs1.reshape(1, H)], axis=1) * BN_S
    Wehs = Weh * BN_S
    behs = beh * BN_S
    Wshs = Wsh * BN_S
    bshs = bsh.reshape(1, H) * BN_S
    Wgp = jnp.concatenate([Wg, jnp.zeros((EP - E, D), Wg.dtype)], axis=0)

    full = lambda shape: pl.BlockSpec(shape, lambda i: (0,) * len(shape))

    lt = pl.pallas_call(
        _logits_body,
        grid_spec=pl.GridSpec(
            grid=(N // TNL,),
            in_specs=[pl.BlockSpec((TNL, D), lambda i: (i, 0)),
                      full((EP, D))],
            out_specs=pl.BlockSpec((EP, TNL), lambda i: (0, i)),
        ),
        out_shape=jax.ShapeDtypeStruct((EP, N), jnp.float32),
        compiler_params=pltpu.CompilerParams(
            dimension_semantics=("arbitrary",),
        ),
    )(combined, Wgp)

    wt = _router_sc(lt)

    grid_spec = pl.GridSpec(
        grid=(N // TN,),
        in_specs=[
            pl.BlockSpec((TN, D), lambda i: (i, 0)),
            pl.BlockSpec((EP, TN), lambda i: (0, i)),
            full(((E + 1) * H, D)), full((1, (E + 1) * H)),
            full((E, H, H)), full((E, H)),
            full((E, O, H)), full((E, O)),
            full((H, H)), full((1, H)),
            full((O, H)), full((1, O)),
        ],
        out_specs=pl.BlockSpec((TN, O), lambda i: (i, 0)),
    )
    return pl.pallas_call(
        _moe_body,
        grid_spec=grid_spec,
        out_shape=jax.ShapeDtypeStruct((N, O), jnp.float32),
        compiler_params=pltpu.CompilerParams(
            dimension_semantics=("parallel",),
        ),
    )(combined, wt, W1c, b1c, Wehs, behs, Weo, beo,
      Wshs, bshs, Wso, bso.reshape(1, O))


def kernel(combined, Wg, We1, be1, Weh, beh, Weo, beo,
           Ws1, bs1, Wsh, bsh, Wso, bso):
    return _run(combined, Wg, We1, be1, Weh, beh, Weo, beo,
                Ws1, bs1, Wsh, bsh, Wso, bso)
